# Initial kernel scaffold; baseline (speedup 1.0000x reference)
import functools

import jax
import jax.numpy as jnp
from jax import lax
from jax.experimental import pallas as pl
from jax.experimental.pallas import tpu as pltpu
from jax.experimental.pallas import tpu_sc as plsc

_NUM_USERS = 30000
_NUM_ITEMS = 70000
_N = _NUM_USERS + _NUM_ITEMS
_E = 1600000
_D = 32
_LAYERS = 3

_L = 16
_NS = 16
_NC = 2

_CHUNK = 2048
_GRP = 128
_NGRP = _CHUNK // _GRP


def _pick_wb(rows_per_tile):
    for wb in (1024, 625, 512, 256, 128, 64, 32, 16):
        if rows_per_tile % wb == 0:
            return wb
    return rows_per_tile


def _build(n, e_pad, n_layers, interpret=False):
    per_tile = e_pad // _NS
    n_chunks = per_tile // _CHUNK
    assert per_tile % _CHUNK == 0
    rows_per_tile = n // _NS
    assert n % _NS == 0
    wb = _pick_wb(rows_per_tile)
    n_wb = rows_per_tile // wb

    mesh = plsc.VectorSubcoreMesh(
        core_axis_name="c", subcore_axis_name="s",
        num_cores=_NC, num_subcores=_NS)

    fdt = jnp.float32
    idt = jnp.int32

    @functools.partial(
        pl.kernel,
        out_type=(
            jax.ShapeDtypeStruct((2 * n, _L), fdt),
            jax.ShapeDtypeStruct((2 * n, _L), fdt),
            jax.ShapeDtypeStruct((2 * n, _L), fdt),
        ),
        mesh=mesh,
        scratch_types=[
            pltpu.VMEM_SHARED((n, _L), fdt),
            pltpu.VMEM((_NGRP, _GRP), idt),
            pltpu.VMEM((_NGRP, _GRP), idt),
            pltpu.VMEM((_NGRP, _GRP), idt),
            pltpu.VMEM((_CHUNK,), fdt),
            pltpu.VMEM((_CHUNK, _L), fdt),
            pltpu.VMEM((_pick_wb(n // _NS), _L), fdt),
            pltpu.VMEM((_pick_wb(n // _NS), _L), fdt),
            pltpu.VMEM((_pick_wb(n // _NS), _L), fdt),
            pltpu.VMEM((_pick_wb(n // _NS), _L), fdt),
            pltpu.VMEM((_pick_wb(n // _NS), _L), fdt),
            pltpu.SemaphoreType.DMA,
            pltpu.SemaphoreType.DMA,
        ],
        interpret=interpret,
    )
    def k(emb0_h, src2_h, dst2_h, val_h, out_h, a_h, b_h,
          acc, src_v, dst_v, gidx_v, val_v, rows_v,
          t_acc, t0, t1, t2, zbuf, gsem, ssem):
        c = lax.axis_index("c")
        s = lax.axis_index("s")
        cn = c * n
        cn_vec = jnp.full((_L,), cn, idt)
        row0 = s * rows_per_tile
        zero16 = jnp.zeros((_L,), fdt)

        def fill_z(i, carry):
            zbuf[i, :] = zero16
            return carry
        lax.fori_loop(0, wb, fill_z, 0)

        def edge_chunk(table_h):
            def body(ch, carry):
                base_e = s * per_tile + ch * _CHUNK
                brow = s * (per_tile // _GRP) + ch * _NGRP
                pltpu.sync_copy(src2_h.at[pl.ds(brow, _NGRP)], src_v)
                pltpu.sync_copy(dst2_h.at[pl.ds(brow, _NGRP)], dst_v)
                pltpu.sync_copy(val_h.at[pl.ds(base_e, _CHUNK)], val_v)

                def gidx_body(t, carry2):
                    i = t >> 3
                    m = (t & 7) * _L
                    gidx_v[i, pl.ds(m, _L)] = src_v[i, pl.ds(m, _L)] + cn_vec
                    return carry2
                lax.fori_loop(0, _CHUNK // _L, gidx_body, 0)

                gd = [
                    pltpu.async_copy(
                        table_h.at[gidx_v.at[j]],
                        rows_v.at[pl.ds(j * _GRP, _GRP)], gsem)
                    for j in range(_NGRP)
                ]
                for d in gd:
                    d.wait()

                def mul_body(g, carry2):
                    v16 = val_v[pl.ds(g * _L, _L)]
                    for kk in range(_L):
                        e = g * _L + kk
                        b = v16.at[jnp.full((_L,), kk, idt)].get(
                            mode="promise_in_bounds")
                        rows_v[e, :] = rows_v[e, :] * b
                    return carry2
                lax.fori_loop(0, _CHUNK // _L, mul_body, 0)

                sd = [
                    pltpu.async_copy(
                        rows_v.at[pl.ds(j * _GRP, _GRP)],
                        acc.at[dst_v.at[j]], ssem, add=True)
                    for j in range(_NGRP)
                ]
                for d in sd:
                    d.wait()
                return carry
            lax.fori_loop(0, n_chunks, body, 0)

        def zero_acc():
            for r in range(n_wb):
                pltpu.sync_copy(zbuf, acc.at[pl.ds(row0 + r * wb, wb)])

        def writeback(dst_h):
            for r in range(n_wb):
                off = row0 + r * wb
                pltpu.sync_copy(acc.at[pl.ds(off, wb)], t_acc)
                pltpu.sync_copy(t_acc, dst_h.at[pl.ds(cn + off, wb)])

        def writeback_mean():
            for r in range(n_wb):
                off = row0 + r * wb
                pltpu.sync_copy(acc.at[pl.ds(off, wb)], t_acc)
                pltpu.sync_copy(emb0_h.at[pl.ds(cn + off, wb)], t0)
                pltpu.sync_copy(a_h.at[pl.ds(cn + off, wb)], t1)
                pltpu.sync_copy(b_h.at[pl.ds(cn + off, wb)], t2)

                def mean_body(i, carry):
                    t0[i, :] = (t0[i, :] + t1[i, :] + t2[i, :]
                                + t_acc[i, :]) * 0.25
                    return carry
                lax.fori_loop(0, wb, mean_body, 0)
                pltpu.sync_copy(t0, out_h.at[pl.ds(cn + off, wb)])

        tables = [emb0_h, a_h, b_h]
        for layer in range(n_layers):
            zero_acc()
            plsc.subcore_barrier()
            edge_chunk(tables[layer])
            plsc.subcore_barrier()
            if layer < n_layers - 1:
                writeback(tables[layer + 1])
            else:
                writeback_mean()

    return k


def _run(n, e, n_layers, user_emb, item_emb, edge_index, edge_vals,
         interpret=False):
    e_pad = ((e + (_NS * _CHUNK) - 1) // (_NS * _CHUNK)) * (_NS * _CHUNK)
    all_emb = jnp.concatenate([user_emb, item_emb], axis=0)
    emb0 = all_emb.reshape(n, 2, _L).transpose(1, 0, 2).reshape(2 * n, _L)
    pad = e_pad - e
    src = jnp.concatenate([edge_index[0], jnp.zeros((pad,), jnp.int32)])
    dst = jnp.concatenate([edge_index[1], jnp.zeros((pad,), jnp.int32)])
    val = jnp.concatenate([edge_vals, jnp.zeros((pad,), jnp.float32)])
    src2 = src.reshape(-1, _GRP)
    dst2 = dst.reshape(-1, _GRP)
    k = _build(n, e_pad, n_layers, interpret=interpret)
    out, _, _ = k(emb0, src2, dst2, val)
    light = out.reshape(2, n, _L).transpose(1, 0, 2).reshape(n, _D)
    return light


@jax.jit
def kernel(user_emb, item_emb, edge_index, edge_vals):
    light = _run(_N, _E, _LAYERS, user_emb, item_emb, edge_index, edge_vals)
    return light[:_NUM_USERS], light[_NUM_USERS:]


# speedup vs baseline: 10.7306x; 10.7306x over previous
import functools

import jax
import jax.numpy as jnp
from jax import lax
from jax.experimental import pallas as pl
from jax.experimental.pallas import tpu as pltpu
from jax.experimental.pallas import tpu_sc as plsc

_NUM_USERS = 30000
_NUM_ITEMS = 70000
_N = _NUM_USERS + _NUM_ITEMS
_E = 1600000
_D = 32
_LAYERS = 3

_L = 16
_NS = 16
_NC = 2

_CHUNK = 1024
_GRP = 128
_NGRP = _CHUNK // _GRP
_WB = 256


def _build(n, e_pad, n_layers, interpret=False):
    per_tile = e_pad // _NS
    n_chunks = per_tile // _CHUNK
    assert per_tile % _CHUNK == 0
    rows_per_tile = n // _NS
    assert n % _NS == 0 and rows_per_tile % _WB == 0
    n_wb = rows_per_tile // _WB

    mesh = plsc.VectorSubcoreMesh(
        core_axis_name="c", subcore_axis_name="s",
        num_cores=_NC, num_subcores=_NS)

    fdt = jnp.float32
    idt = jnp.int32

    @functools.partial(
        pl.kernel,
        out_type=(
            jax.ShapeDtypeStruct((2 * n, _L), fdt),
            jax.ShapeDtypeStruct((2 * n, _L), fdt),
            jax.ShapeDtypeStruct((2 * n, _L), fdt),
        ),
        mesh=mesh,
        scratch_types=[
            pltpu.VMEM_SHARED((n, _L), fdt),
            pltpu.VMEM((_NGRP, _GRP), idt),
            pltpu.VMEM((_NGRP, _GRP), idt),
            pltpu.VMEM((_NGRP, _GRP), idt),
            pltpu.VMEM((_CHUNK,), fdt),
            pltpu.VMEM((_CHUNK, _L), fdt),
            pltpu.SemaphoreType.DMA,
            pltpu.SemaphoreType.DMA,
        ],
        compiler_params=pltpu.CompilerParams(use_tc_tiling_on_sc=False),
        interpret=interpret,
    )
    def k(emb0_h, src2_h, dst2_h, val_h, out_h, a_h, b_h,
          acc, src_v, dst_v, gidx_v, val_v, rows_v, gsem, ssem):
        c = lax.axis_index("c")
        s = lax.axis_index("s")
        cn = c * n
        cn_vec = jnp.full((_L,), cn, idt)
        row0 = s * rows_per_tile
        zero16 = jnp.zeros((_L,), fdt)

        def edge_chunk(table_h):
            def body(ch, carry):
                base_e = s * per_tile + ch * _CHUNK
                brow = s * (per_tile // _GRP) + ch * _NGRP
                pltpu.sync_copy(src2_h.at[pl.ds(brow, _NGRP)], src_v)
                pltpu.sync_copy(dst2_h.at[pl.ds(brow, _NGRP)], dst_v)
                pltpu.sync_copy(val_h.at[pl.ds(base_e, _CHUNK)], val_v)

                def gidx_body(t, carry2):
                    i = t >> 3
                    m = (t & 7) * _L
                    gidx_v[i, pl.ds(m, _L)] = src_v[i, pl.ds(m, _L)] + cn_vec
                    return carry2
                lax.fori_loop(0, _CHUNK // _L, gidx_body, 0)

                gd = [
                    pltpu.async_copy(
                        table_h.at[gidx_v.at[j]],
                        rows_v.at[pl.ds(j * _GRP, _GRP)], gsem)
                    for j in range(_NGRP)
                ]
                for d in gd:
                    d.wait()

                def mul_body(g, carry2):
                    v16 = val_v[pl.ds(g * _L, _L)]
                    for kk in range(_L):
                        e = g * _L + kk
                        b = v16.at[jnp.full((_L,), kk, idt)].get(
                            mode="promise_in_bounds")
                        rows_v[e, :] = rows_v[e, :] * b
                    return carry2
                lax.fori_loop(0, _CHUNK // _L, mul_body, 0)

                sd = [
                    pltpu.async_copy(
                        rows_v.at[pl.ds(j * _GRP, _GRP)],
                        acc.at[dst_v.at[j]], ssem, add=True)
                    for j in range(_NGRP)
                ]
                for d in sd:
                    d.wait()
                return carry
            lax.fori_loop(0, n_chunks, body, 0)

        def zero_acc():
            def fill_z(i, carry):
                rows_v[i, :] = zero16
                return carry
            lax.fori_loop(0, _WB, fill_z, 0)
            for r in range(n_wb):
                pltpu.sync_copy(rows_v.at[pl.ds(0, _WB)],
                                acc.at[pl.ds(row0 + r * _WB, _WB)])

        def writeback(dst_h):
            for r in range(n_wb):
                off = row0 + r * _WB
                pltpu.sync_copy(acc.at[pl.ds(off, _WB)],
                                rows_v.at[pl.ds(0, _WB)])
                pltpu.sync_copy(rows_v.at[pl.ds(0, _WB)],
                                dst_h.at[pl.ds(cn + off, _WB)])

        def writeback_mean():
            for r in range(n_wb):
                off = row0 + r * _WB
                pltpu.sync_copy(acc.at[pl.ds(off, _WB)],
                                rows_v.at[pl.ds(0, _WB)])
                pltpu.sync_copy(emb0_h.at[pl.ds(cn + off, _WB)],
                                rows_v.at[pl.ds(_WB, _WB)])
                pltpu.sync_copy(a_h.at[pl.ds(cn + off, _WB)],
                                rows_v.at[pl.ds(2 * _WB, _WB)])
                pltpu.sync_copy(b_h.at[pl.ds(cn + off, _WB)],
                                rows_v.at[pl.ds(3 * _WB, _WB)])

                def mean_body(i, carry):
                    rows_v[i, :] = (rows_v[i, :] + rows_v[_WB + i, :]
                                    + rows_v[2 * _WB + i, :]
                                    + rows_v[3 * _WB + i, :]) * 0.25
                    return carry
                lax.fori_loop(0, _WB, mean_body, 0)
                pltpu.sync_copy(rows_v.at[pl.ds(0, _WB)],
                                out_h.at[pl.ds(cn + off, _WB)])

        tables = [emb0_h, a_h, b_h]
        for layer in range(n_layers):
            zero_acc()
            plsc.subcore_barrier()
            edge_chunk(tables[layer])
            plsc.subcore_barrier()
            if layer < n_layers - 1:
                writeback(tables[layer + 1])
            else:
                writeback_mean()

    return k


def _run(n, e, n_layers, user_emb, item_emb, edge_index, edge_vals,
         interpret=False):
    e_pad = ((e + (_NS * _CHUNK) - 1) // (_NS * _CHUNK)) * (_NS * _CHUNK)
    n_pad = ((n + (_NS * _WB) - 1) // (_NS * _WB)) * (_NS * _WB)
    all_emb = jnp.concatenate([user_emb, item_emb], axis=0)
    all_emb = jnp.pad(all_emb, ((0, n_pad - n), (0, 0)))
    emb0 = all_emb.reshape(n_pad, 2, _L).transpose(1, 0, 2).reshape(
        2 * n_pad, _L)
    pad = e_pad - e
    src = jnp.concatenate([edge_index[0], jnp.zeros((pad,), jnp.int32)])
    dst = jnp.concatenate([edge_index[1], jnp.zeros((pad,), jnp.int32)])
    val = jnp.concatenate([edge_vals, jnp.zeros((pad,), jnp.float32)])
    src2 = src.reshape(-1, _GRP)
    dst2 = dst.reshape(-1, _GRP)
    k = _build(n_pad, e_pad, n_layers, interpret=interpret)
    out, _, _ = k(emb0, src2, dst2, val)
    light = out.reshape(2, n_pad, _L)[:, :n, :].transpose(1, 0, 2).reshape(
        n, _D)
    return light


@jax.jit
def kernel(user_emb, item_emb, edge_index, edge_vals):
    light = _run(_N, _E, _LAYERS, user_emb, item_emb, edge_index, edge_vals)
    return light[:_NUM_USERS], light[_NUM_USERS:]
